# baseline (speedup 1.0000x reference)
import functools

import jax
import jax.numpy as jnp
from jax import lax
from jax.experimental import pallas as pl
from jax.experimental.pallas import tpu as pltpu
from jax.experimental.pallas import tpu_sc as plsc

NC = 2
NS = 16
NW = NC * NS
L = 16

_MESH = plsc.VectorSubcoreMesh(core_axis_name="c", subcore_axis_name="s")


@functools.lru_cache(maxsize=None)
def _deg_call(E, N):
    epw = E // NW
    nvec = epw // L
    nhz = N // L

    def body(src_hbm, dst_hbm, hs_out, hd_out, idx_v, hist):
        cid = lax.axis_index("c")
        sid = lax.axis_index("s")
        wid = cid * NS + sid
        zeros16 = jnp.zeros((L,), jnp.float32)
        ones16 = jnp.ones((L,), jnp.float32)

        def run(ind_hbm, out_hbm):
            def zero_it(i, c):
                hist[pl.ds(i * L, L)] = zeros16
                return c
            lax.fori_loop(0, nhz, zero_it, 0)
            pltpu.sync_copy(ind_hbm.at[pl.ds(wid * epw, epw)], idx_v)

            def acc(i, c):
                idx = idx_v[pl.ds(i * L, L)]
                plsc.addupdate_scatter(hist, [idx], ones16)
                return c
            lax.fori_loop(0, nvec, acc, 0)
            pltpu.sync_copy(hist, out_hbm.at[wid, 0])

        run(src_hbm, hs_out)
        run(dst_hbm, hd_out)

    return pl.kernel(
        body,
        out_type=[
            jax.ShapeDtypeStruct((NW, 1, N), jnp.float32),
            jax.ShapeDtypeStruct((NW, 1, N), jnp.float32),
        ],
        mesh=_MESH,
        scratch_types=[
            pltpu.VMEM((epw,), jnp.int32),
            pltpu.VMEM((N,), jnp.float32),
        ],
        compiler_params=pltpu.CompilerParams(needs_layout_passes=False),
    )


_CH = 64
_NBUF = 4


@functools.lru_cache(maxsize=None)
def _agg_call(E, N, D):
    epw = E // NW
    nch = epw // _CH
    ng = nch // _NBUF
    Npad = (N // 128 + 1) * 128
    rpt = Npad // NS

    def body(h_hbm, src_hbm, dst_hbm, zz_hbm, out_hbm,
             sidx, d0, d1, d2, d3, r0, r1, r2, r3,
             agg_sh, g0, g1, g2, g3, s0, s1, s2, s3, y0, y1, y2, y3):
        rows = (r0, r1, r2, r3)
        didx = (d0, d1, d2, d3)
        gsem = (g0, g1, g2, g3)
        ssem = (s0, s1, s2, s3)
        ysem = (y0, y1, y2, y3)
        cid = lax.axis_index("c")
        sid = lax.axis_index("s")
        wid = cid * NS + sid

        def gather(i, k):
            return pltpu.make_async_copy(
                h_hbm.at[sidx.at[pl.ds(i * _CH, _CH)]], rows[k], gsem[k])

        def dload(i, k):
            return pltpu.make_async_copy(
                dst_hbm.at[pl.ds(wid * epw + i * _CH, _CH)], didx[k], ysem[k])

        def scatter(k):
            return pltpu.make_async_copy(rows[k], agg_sh.at[didx[k]], ssem[k])

        pltpu.sync_copy(src_hbm.at[pl.ds(wid * epw, epw)], sidx)
        for k in range(2):
            dload(k, k).start()
            gather(k, k).start()
        pltpu.sync_copy(zz_hbm.at[pl.ds(sid * rpt, rpt)],
                        agg_sh.at[pl.ds(sid * rpt, rpt)])
        plsc.subcore_barrier()

        def group(g, c):
            for k in range(_NBUF):
                i = g * _NBUF + k
                j = i + 2
                kk = (k + 2) % _NBUF
                gather(i, k).wait()
                dload(i, k).wait()
                scatter(k).start(add=True)
                @pl.when(i >= 2)
                def _():
                    scatter(kk).wait()
                @pl.when(j < nch)
                def _():
                    dload(j, kk).start()
                    gather(j, kk).start()
            return c
        lax.fori_loop(0, ng, group, 0)

        scatter((nch - 2) % _NBUF).wait()
        scatter((nch - 1) % _NBUF).wait()
        plsc.subcore_barrier()
        pltpu.sync_copy(agg_sh.at[pl.ds(sid * rpt, rpt)],
                        out_hbm.at[cid, pl.ds(sid * rpt, rpt)])

    return pl.kernel(
        body,
        out_type=jax.ShapeDtypeStruct((NC, Npad, D), jnp.float32),
        mesh=_MESH,
        scratch_types=[
            pltpu.VMEM((epw,), jnp.int32),
            pltpu.VMEM((_CH,), jnp.int32),
            pltpu.VMEM((_CH,), jnp.int32),
            pltpu.VMEM((_CH,), jnp.int32),
            pltpu.VMEM((_CH,), jnp.int32),
            pltpu.VMEM((_CH, D), jnp.float32),
            pltpu.VMEM((_CH, D), jnp.float32),
            pltpu.VMEM((_CH, D), jnp.float32),
            pltpu.VMEM((_CH, D), jnp.float32),
            pltpu.VMEM_SHARED((Npad, D), jnp.float32),
        ] + [pltpu.SemaphoreType.DMA] * 12,
        compiler_params=pltpu.CompilerParams(needs_layout_passes=False),
    )


def _prep_body(x_ref, hs_ref, hd_ref, h1_ref, ns_ref, nd_ref):
    ds = jnp.sum(hs_ref[...], axis=1, keepdims=True)
    dd = jnp.sum(hd_ref[...], axis=1, keepdims=True)
    ns = jnp.where(ds > 0, lax.rsqrt(ds), 0.0)
    nd = jnp.where(dd > 0, lax.rsqrt(dd), 0.0)
    ns_ref[...] = ns
    nd_ref[...] = nd
    h1_ref[...] = x_ref[...] * ns


@functools.lru_cache(maxsize=None)
def _prep_call(N, D, R=400):
    grid = N // R
    return pl.pallas_call(
        _prep_body,
        grid=(grid,),
        in_specs=[
            pl.BlockSpec((R, D), lambda i: (i, 0)),
            pl.BlockSpec((R, NW), lambda i: (i, 0)),
            pl.BlockSpec((R, NW), lambda i: (i, 0)),
        ],
        out_specs=[
            pl.BlockSpec((R, D), lambda i: (i, 0)),
            pl.BlockSpec((R, 1), lambda i: (i, 0)),
            pl.BlockSpec((R, 1), lambda i: (i, 0)),
        ],
        out_shape=[
            jax.ShapeDtypeStruct((N, D), jnp.float32),
            jax.ShapeDtypeStruct((N, 1), jnp.float32),
            jax.ShapeDtypeStruct((N, 1), jnp.float32),
        ],
    )


def _dense_body(apply_tanh, agg_ref, nd_ref, ns_ref, w_ref, b_ref, out_ref):
    a = (agg_ref[0] + agg_ref[1]) * nd_ref[...]
    y = jnp.dot(a, w_ref[...], preferred_element_type=jnp.float32,
                precision=lax.Precision.HIGHEST) + b_ref[...]
    if apply_tanh:
        y = jnp.tanh(y) * ns_ref[...]
    out_ref[...] = y


@functools.lru_cache(maxsize=None)
def _dense_call(N, D, H, apply_tanh, R=400):
    grid = N // R
    return pl.pallas_call(
        functools.partial(_dense_body, apply_tanh),
        grid=(grid,),
        in_specs=[
            pl.BlockSpec((NC, R, D), lambda i: (0, i, 0)),
            pl.BlockSpec((R, 1), lambda i: (i, 0)),
            pl.BlockSpec((R, 1), lambda i: (i, 0)),
            pl.BlockSpec((D, H), lambda i: (0, 0)),
            pl.BlockSpec((1, H), lambda i: (0, 0)),
        ],
        out_specs=pl.BlockSpec((R, H), lambda i: (i, 0)),
        out_shape=jax.ShapeDtypeStruct((N, H), jnp.float32),
    )


def kernel(x, edge_index, W1, b1, W2, b2):
    N, D = x.shape
    H = W1.shape[1]
    E = edge_index.shape[1]
    src = edge_index[0]
    dst = edge_index[1]

    hs, hd = _deg_call(E, N)(src, dst)
    h1, ns, nd = _prep_call(N, D)(x, hs[:, 0, :].T, hd[:, 0, :].T)

    Npad = (N // 128 + 1) * 128
    step = _NBUF * _CH
    epw_pad = -(-(E // NW) // step) * step
    Epad = epw_pad * NW
    if Epad != E:
        pad = jnp.zeros((Epad - E,), jnp.int32)
        src_p = jnp.concatenate([src, pad])
        dpad = N + jnp.arange(Epad - E, dtype=jnp.int32) % (Npad - N)
        dst_p = jnp.concatenate([dst, dpad])
    else:
        src_p, dst_p = src, dst

    zz = jnp.zeros((Npad, D), jnp.float32)
    agg1 = _agg_call(Epad, N, D)(h1, src_p, dst_p, zz)
    h2 = _dense_call(N, D, H, True)(agg1, nd, ns, W1, b1[None])

    agg2 = _agg_call(Epad, N, H)(h2, src_p, dst_p, zz)
    half = D // 2
    W2e = jnp.concatenate([W2[:, half:], -W2[:, :half]], axis=1)
    b2e = jnp.concatenate([b2[half:], -b2[:half]])
    out = _dense_call(N, H, D, False)(agg2, nd, ns, W2e, b2e[None])
    return out


# speedup vs baseline: 3.1626x; 3.1626x over previous
import functools

import jax
import jax.numpy as jnp
from jax import lax
from jax.experimental import pallas as pl
from jax.experimental.pallas import tpu as pltpu
from jax.experimental.pallas import tpu_sc as plsc

NC = 2
NS = 16
NW = NC * NS
L = 16

_MESH = plsc.VectorSubcoreMesh(core_axis_name="c", subcore_axis_name="s")


@functools.lru_cache(maxsize=None)
def _deg_call(E, N):
    epw = E // NW
    nvec = epw // L
    nhz = N // L

    def body(src_hbm, dst_hbm, hs_out, hd_out, idx_v, hist):
        cid = lax.axis_index("c")
        sid = lax.axis_index("s")
        wid = cid * NS + sid
        zeros16 = jnp.zeros((L,), jnp.float32)
        ones16 = jnp.ones((L,), jnp.float32)

        def run(ind_hbm, out_hbm):
            def zero_it(i, c):
                hist[pl.ds(i * L, L)] = zeros16
                return c
            lax.fori_loop(0, nhz, zero_it, 0)
            pltpu.sync_copy(ind_hbm.at[pl.ds(wid * epw, epw)], idx_v)

            def acc(i, c):
                idx = idx_v[pl.ds(i * L, L)]
                plsc.addupdate_scatter(hist, [idx], ones16)
                return c
            lax.fori_loop(0, nvec, acc, 0)
            pltpu.sync_copy(hist, out_hbm.at[wid, 0])

        run(src_hbm, hs_out)
        run(dst_hbm, hd_out)

    return pl.kernel(
        body,
        out_type=[
            jax.ShapeDtypeStruct((NW, 1, N), jnp.float32),
            jax.ShapeDtypeStruct((NW, 1, N), jnp.float32),
        ],
        mesh=_MESH,
        scratch_types=[
            pltpu.VMEM((epw,), jnp.int32),
            pltpu.VMEM((N,), jnp.float32),
        ],
        compiler_params=pltpu.CompilerParams(needs_layout_passes=False),
    )


_CH = 64
_NBUF = 4


@functools.lru_cache(maxsize=None)
def _agg_call(E, N, D):
    epw = E // NW
    nch = epw // _CH
    ng = nch // _NBUF
    Npad = (N // 128 + 1) * 128
    rpt = Npad // NS

    def body(h_hbm, src_hbm, dst_hbm, zz_hbm, out_hbm,
             sidx, d0, d1, d2, d3, r0, r1, r2, r3,
             agg_sh, g0, g1, g2, g3, s0, s1, s2, s3, y0, y1, y2, y3):
        rows = (r0, r1, r2, r3)
        didx = (d0, d1, d2, d3)
        gsem = (g0, g1, g2, g3)
        ssem = (s0, s1, s2, s3)
        ysem = (y0, y1, y2, y3)
        cid = lax.axis_index("c")
        sid = lax.axis_index("s")
        wid = cid * NS + sid

        def gather(i, k):
            return pltpu.make_async_copy(
                h_hbm.at[sidx.at[pl.ds(i * _CH, _CH)]], rows[k], gsem[k])

        def dload(i, k):
            return pltpu.make_async_copy(
                dst_hbm.at[pl.ds(wid * epw + i * _CH, _CH)], didx[k], ysem[k])

        def scatter(k):
            return pltpu.make_async_copy(rows[k], agg_sh.at[didx[k]], ssem[k])

        pltpu.sync_copy(src_hbm.at[pl.ds(wid * epw, epw)], sidx)
        for k in range(2):
            dload(k, k).start()
            gather(k, k).start()
        pltpu.sync_copy(zz_hbm.at[pl.ds(sid * rpt, rpt)],
                        agg_sh.at[pl.ds(sid * rpt, rpt)])
        plsc.subcore_barrier()

        def group(g, c):
            for k in range(_NBUF):
                i = g * _NBUF + k
                j = i + 2
                kk = (k + 2) % _NBUF
                gather(i, k).wait()
                dload(i, k).wait()
                scatter(k).start(add=True)
                @pl.when(i >= 2)
                def _():
                    scatter(kk).wait()
                @pl.when(j < nch)
                def _():
                    dload(j, kk).start()
                    gather(j, kk).start()
            return c
        lax.fori_loop(0, ng, group, 0)

        scatter((nch - 2) % _NBUF).wait()
        scatter((nch - 1) % _NBUF).wait()
        plsc.subcore_barrier()
        pltpu.sync_copy(agg_sh.at[pl.ds(sid * rpt, rpt)],
                        out_hbm.at[cid, pl.ds(sid * rpt, rpt)])

    return pl.kernel(
        body,
        out_type=jax.ShapeDtypeStruct((NC, Npad, D), jnp.float32),
        mesh=_MESH,
        scratch_types=[
            pltpu.VMEM((epw,), jnp.int32),
            pltpu.VMEM((_CH,), jnp.int32),
            pltpu.VMEM((_CH,), jnp.int32),
            pltpu.VMEM((_CH,), jnp.int32),
            pltpu.VMEM((_CH,), jnp.int32),
            pltpu.VMEM((_CH, D), jnp.float32),
            pltpu.VMEM((_CH, D), jnp.float32),
            pltpu.VMEM((_CH, D), jnp.float32),
            pltpu.VMEM((_CH, D), jnp.float32),
            pltpu.VMEM_SHARED((Npad, D), jnp.float32),
        ] + [pltpu.SemaphoreType.DMA] * 12,
        compiler_params=pltpu.CompilerParams(needs_layout_passes=False),
    )


def _prep_body(x_ref, hs_ref, hd_ref, h1_ref, ns_ref, nd_ref):
    ds = jnp.sum(hs_ref[...], axis=1, keepdims=True)
    dd = jnp.sum(hd_ref[...], axis=1, keepdims=True)
    ns = jnp.where(ds > 0, lax.rsqrt(ds), 0.0)
    nd = jnp.where(dd > 0, lax.rsqrt(dd), 0.0)
    ns_ref[...] = ns
    nd_ref[...] = nd
    h1_ref[...] = x_ref[...] * ns


@functools.lru_cache(maxsize=None)
def _prep_call(N, D, R=400):
    grid = N // R
    return pl.pallas_call(
        _prep_body,
        grid=(grid,),
        in_specs=[
            pl.BlockSpec((R, D), lambda i: (i, 0)),
            pl.BlockSpec((R, NW), lambda i: (i, 0)),
            pl.BlockSpec((R, NW), lambda i: (i, 0)),
        ],
        out_specs=[
            pl.BlockSpec((R, D), lambda i: (i, 0)),
            pl.BlockSpec((R, 1), lambda i: (i, 0)),
            pl.BlockSpec((R, 1), lambda i: (i, 0)),
        ],
        out_shape=[
            jax.ShapeDtypeStruct((N, D), jnp.float32),
            jax.ShapeDtypeStruct((N, 1), jnp.float32),
            jax.ShapeDtypeStruct((N, 1), jnp.float32),
        ],
    )


def _dense_body(apply_tanh, agg_ref, nd_ref, ns_ref, w_ref, b_ref, out_ref):
    a = (agg_ref[0] + agg_ref[1]) * nd_ref[...]
    y = jnp.dot(a, w_ref[...], preferred_element_type=jnp.float32,
                precision=lax.Precision.HIGHEST) + b_ref[...]
    if apply_tanh:
        y = jnp.tanh(y) * ns_ref[...]
    out_ref[...] = y


@functools.lru_cache(maxsize=None)
def _dense_call(N, D, H, apply_tanh, R=400):
    grid = N // R
    return pl.pallas_call(
        functools.partial(_dense_body, apply_tanh),
        grid=(grid,),
        in_specs=[
            pl.BlockSpec((NC, R, D), lambda i: (0, i, 0)),
            pl.BlockSpec((R, 1), lambda i: (i, 0)),
            pl.BlockSpec((R, 1), lambda i: (i, 0)),
            pl.BlockSpec((D, H), lambda i: (0, 0)),
            pl.BlockSpec((1, H), lambda i: (0, 0)),
        ],
        out_specs=pl.BlockSpec((R, H), lambda i: (i, 0)),
        out_shape=jax.ShapeDtypeStruct((N, H), jnp.float32),
    )


def kernel(x, edge_index, W1, b1, W2, b2):
    N, D = x.shape
    H = W1.shape[1]
    E = edge_index.shape[1]
    src = edge_index[0]
    dst = edge_index[1]

    hs, hd = _deg_call(E, N)(src, dst)
    h1, ns, nd = _prep_call(N, D)(x, hs[:, 0, :].T, hd[:, 0, :].T)

    Npad = (N // 128 + 1) * 128
    step = _NBUF * _CH
    epw_pad = -(-(E // NW) // step) * step
    Epad = epw_pad * NW
    if Epad != E:
        ppw = epw_pad - E // NW
        iota = jnp.arange(ppw * NW, dtype=jnp.int32).reshape(NW, ppw)
        spad = iota % N
        dpad = N + iota % (Npad - N)
        src_p = jnp.concatenate([src.reshape(NW, -1), spad], axis=1).reshape(-1)
        dst_p = jnp.concatenate([dst.reshape(NW, -1), dpad], axis=1).reshape(-1)
    else:
        src_p, dst_p = src, dst

    zz = jnp.zeros((Npad, D), jnp.float32)
    agg1 = _agg_call(Epad, N, D)(h1, src_p, dst_p, zz)
    h2 = _dense_call(N, D, H, True)(agg1, nd, ns, W1, b1[None])

    agg2 = _agg_call(Epad, N, H)(h2, src_p, dst_p, zz)
    half = D // 2
    W2e = jnp.concatenate([W2[:, half:], -W2[:, :half]], axis=1)
    b2e = jnp.concatenate([b2[half:], -b2[:half]])
    out = _dense_call(N, H, D, False)(agg2, nd, ns, W2e, b2e[None])
    return out


# speedup vs baseline: 3.2970x; 1.0425x over previous
import functools

import jax
import jax.numpy as jnp
from jax import lax
from jax.experimental import pallas as pl
from jax.experimental.pallas import tpu as pltpu
from jax.experimental.pallas import tpu_sc as plsc

NC = 2
NS = 16
NW = NC * NS
L = 16

_MESH = plsc.VectorSubcoreMesh(core_axis_name="c", subcore_axis_name="s")


@functools.lru_cache(maxsize=None)
def _deg_call(E, N):
    epw = E // NW
    nvec = epw // L
    nhz = N // L

    def body(src_hbm, dst_hbm, hs_out, hd_out, idx_v, hist):
        cid = lax.axis_index("c")
        sid = lax.axis_index("s")
        wid = cid * NS + sid
        zeros16 = jnp.zeros((L,), jnp.float32)
        ones16 = jnp.ones((L,), jnp.float32)

        def run(ind_hbm, out_hbm):
            def zero_it(i, c):
                hist[pl.ds(i * L, L)] = zeros16
                return c
            lax.fori_loop(0, nhz, zero_it, 0)
            pltpu.sync_copy(ind_hbm.at[pl.ds(wid * epw, epw)], idx_v)

            def acc(i, c):
                idx = idx_v[pl.ds(i * L, L)]
                plsc.addupdate_scatter(hist, [idx], ones16)
                return c
            lax.fori_loop(0, nvec, acc, 0)
            pltpu.sync_copy(hist, out_hbm.at[wid, 0])

        run(src_hbm, hs_out)
        run(dst_hbm, hd_out)

    return pl.kernel(
        body,
        out_type=[
            jax.ShapeDtypeStruct((NW, 1, N), jnp.float32),
            jax.ShapeDtypeStruct((NW, 1, N), jnp.float32),
        ],
        mesh=_MESH,
        scratch_types=[
            pltpu.VMEM((epw,), jnp.int32),
            pltpu.VMEM((N,), jnp.float32),
        ],
        compiler_params=pltpu.CompilerParams(needs_layout_passes=False),
    )


_CH = 80
_NBUF = 4


@functools.lru_cache(maxsize=None)
def _agg_call(E, N, D):
    epw = E // NW
    nch = epw // _CH
    ng = nch // _NBUF
    rem = nch - ng * _NBUF
    assert rem <= 2
    Npad = (N // 128 + 1) * 128
    rpt = Npad // NS

    def body(h_hbm, src_hbm, dst_hbm, zz_hbm, out_hbm,
             s0, s1, s2, s3, d0, d1, d2, d3, r0, r1, r2, r3,
             agg_sh, *sems):
        rows = (r0, r1, r2, r3)
        sidx = (s0, s1, s2, s3)
        didx = (d0, d1, d2, d3)
        gsem, ssem, xsem, ysem = sems[0:4], sems[4:8], sems[8:12], sems[12:16]
        cid = lax.axis_index("c")
        sid = lax.axis_index("s")
        wid = cid * NS + sid

        def sload(i, k):
            return pltpu.make_async_copy(
                src_hbm.at[pl.ds(wid * epw + i * _CH, _CH)], sidx[k], xsem[k])

        def dload(i, k):
            return pltpu.make_async_copy(
                dst_hbm.at[pl.ds(wid * epw + i * _CH, _CH)], didx[k], ysem[k])

        def gather(i, k):
            return pltpu.make_async_copy(h_hbm.at[sidx[k]], rows[k], gsem[k])

        def scatter(k):
            return pltpu.make_async_copy(rows[k], agg_sh.at[didx[k]], ssem[k])

        for k in range(_NBUF):
            sload(k, k).start()
        for k in range(2):
            dload(k, k).start()
        for k in range(2):
            sload(k, k).wait()
            gather(k, k).start()
        pltpu.sync_copy(zz_hbm.at[pl.ds(sid * rpt, rpt)],
                        agg_sh.at[pl.ds(sid * rpt, rpt)])
        plsc.subcore_barrier()

        def slot(i, k):
            gather(i, k).wait()
            dload(i, k).wait()
            scatter(k).start(add=True)
            kk = (k + 2) % _NBUF
            @pl.when(i >= 2)
            def _():
                scatter(kk).wait()
            @pl.when(i + 2 < nch)
            def _():
                dload(i + 2, kk).start()
                sload(i + 2, kk).wait()
                gather(i + 2, kk).start()
            @pl.when(i + 4 < nch)
            def _():
                sload(i + 4, k).start()

        def group(g, c):
            for k in range(_NBUF):
                slot(g * _NBUF + k, k)
            return c
        lax.fori_loop(0, ng, group, 0)
        for t in range(rem):
            slot(ng * _NBUF + t, t)

        for i in range(nch - 2, nch):
            scatter(i % _NBUF).wait()
        plsc.subcore_barrier()
        pltpu.sync_copy(agg_sh.at[pl.ds(sid * rpt, rpt)],
                        out_hbm.at[cid, pl.ds(sid * rpt, rpt)])

    return pl.kernel(
        body,
        out_type=jax.ShapeDtypeStruct((NC, Npad, D), jnp.float32),
        mesh=_MESH,
        scratch_types=[pltpu.VMEM((_CH,), jnp.int32)] * 8
        + [pltpu.VMEM((_CH, D), jnp.float32)] * 4
        + [pltpu.VMEM_SHARED((Npad, D), jnp.float32)]
        + [pltpu.SemaphoreType.DMA] * 16,
        compiler_params=pltpu.CompilerParams(needs_layout_passes=False),
    )


def _prep_body(x_ref, hs_ref, hd_ref, h1_ref, ns_ref, nd_ref):
    ds = jnp.sum(hs_ref[...], axis=1, keepdims=True)
    dd = jnp.sum(hd_ref[...], axis=1, keepdims=True)
    ns = jnp.where(ds > 0, lax.rsqrt(ds), 0.0)
    nd = jnp.where(dd > 0, lax.rsqrt(dd), 0.0)
    ns_ref[...] = ns
    nd_ref[...] = nd
    h1_ref[...] = x_ref[...] * ns


@functools.lru_cache(maxsize=None)
def _prep_call(N, D, R=400):
    grid = N // R
    return pl.pallas_call(
        _prep_body,
        grid=(grid,),
        in_specs=[
            pl.BlockSpec((R, D), lambda i: (i, 0)),
            pl.BlockSpec((R, NW), lambda i: (i, 0)),
            pl.BlockSpec((R, NW), lambda i: (i, 0)),
        ],
        out_specs=[
            pl.BlockSpec((R, D), lambda i: (i, 0)),
            pl.BlockSpec((R, 1), lambda i: (i, 0)),
            pl.BlockSpec((R, 1), lambda i: (i, 0)),
        ],
        out_shape=[
            jax.ShapeDtypeStruct((N, D), jnp.float32),
            jax.ShapeDtypeStruct((N, 1), jnp.float32),
            jax.ShapeDtypeStruct((N, 1), jnp.float32),
        ],
    )


def _dense_body(apply_tanh, agg_ref, nd_ref, ns_ref, w_ref, b_ref, out_ref):
    a = (agg_ref[0] + agg_ref[1]) * nd_ref[...]
    y = jnp.dot(a, w_ref[...], preferred_element_type=jnp.float32,
                precision=lax.Precision.HIGHEST) + b_ref[...]
    if apply_tanh:
        y = jnp.tanh(y) * ns_ref[...]
    out_ref[...] = y


@functools.lru_cache(maxsize=None)
def _dense_call(N, D, H, apply_tanh, R=400):
    grid = N // R
    return pl.pallas_call(
        functools.partial(_dense_body, apply_tanh),
        grid=(grid,),
        in_specs=[
            pl.BlockSpec((NC, R, D), lambda i: (0, i, 0)),
            pl.BlockSpec((R, 1), lambda i: (i, 0)),
            pl.BlockSpec((R, 1), lambda i: (i, 0)),
            pl.BlockSpec((D, H), lambda i: (0, 0)),
            pl.BlockSpec((1, H), lambda i: (0, 0)),
        ],
        out_specs=pl.BlockSpec((R, H), lambda i: (i, 0)),
        out_shape=jax.ShapeDtypeStruct((N, H), jnp.float32),
    )


def kernel(x, edge_index, W1, b1, W2, b2):
    N, D = x.shape
    H = W1.shape[1]
    E = edge_index.shape[1]
    src = edge_index[0]
    dst = edge_index[1]

    hs, hd = _deg_call(E, N)(src, dst)
    h1, ns, nd = _prep_call(N, D)(x, hs[:, 0, :].T, hd[:, 0, :].T)

    Npad = (N // 128 + 1) * 128
    epw_pad = -(-(E // NW) // _CH) * _CH
    while (epw_pad // _CH) % _NBUF > 2:
        epw_pad += _CH
    Epad = epw_pad * NW
    if Epad != E:
        ppw = epw_pad - E // NW
        iota = jnp.arange(ppw * NW, dtype=jnp.int32).reshape(NW, ppw)
        spad = iota % N
        dpad = N + iota % (Npad - N)
        src_p = jnp.concatenate([src.reshape(NW, -1), spad], axis=1).reshape(-1)
        dst_p = jnp.concatenate([dst.reshape(NW, -1), dpad], axis=1).reshape(-1)
    else:
        src_p, dst_p = src, dst

    zz = jnp.zeros((Npad, D), jnp.float32)
    agg1 = _agg_call(Epad, N, D)(h1, src_p, dst_p, zz)
    h2 = _dense_call(N, D, H, True)(agg1, nd, ns, W1, b1[None])

    agg2 = _agg_call(Epad, N, H)(h2, src_p, dst_p, zz)
    half = D // 2
    W2e = jnp.concatenate([W2[:, half:], -W2[:, :half]], axis=1)
    b2e = jnp.concatenate([b2[half:], -b2[:half]])
    out = _dense_call(N, H, D, False)(agg2, nd, ns, W2e, b2e[None])
    return out


# speedup vs baseline: 3.3376x; 1.0123x over previous
import functools

import jax
import jax.numpy as jnp
from jax import lax
from jax.experimental import pallas as pl
from jax.experimental.pallas import tpu as pltpu
from jax.experimental.pallas import tpu_sc as plsc

NC = 2
NS = 16
NW = NC * NS
L = 16

_MESH = plsc.VectorSubcoreMesh(core_axis_name="c", subcore_axis_name="s")


@functools.lru_cache(maxsize=None)
def _deg_call(E, N):
    epw = E // NW
    nvec = epw // L
    nhz = N // L

    def body(src_hbm, dst_hbm, hs_out, hd_out, idx_v, hist):
        cid = lax.axis_index("c")
        sid = lax.axis_index("s")
        wid = cid * NS + sid
        zeros16 = jnp.zeros((L,), jnp.float32)
        ones16 = jnp.ones((L,), jnp.float32)

        def run(ind_hbm, out_hbm):
            def zero_it(i, c):
                hist[pl.ds(i * L, L)] = zeros16
                return c
            lax.fori_loop(0, nhz, zero_it, 0)
            pltpu.sync_copy(ind_hbm.at[pl.ds(wid * epw, epw)], idx_v)

            def acc(i, c):
                idx = idx_v[pl.ds(i * L, L)]
                plsc.addupdate_scatter(hist, [idx], ones16)
                return c
            lax.fori_loop(0, nvec, acc, 0)
            pltpu.sync_copy(hist, out_hbm.at[wid, 0])

        run(src_hbm, hs_out)
        run(dst_hbm, hd_out)

    return pl.kernel(
        body,
        out_type=[
            jax.ShapeDtypeStruct((NW, 1, N), jnp.float32),
            jax.ShapeDtypeStruct((NW, 1, N), jnp.float32),
        ],
        mesh=_MESH,
        scratch_types=[
            pltpu.VMEM((epw,), jnp.int32),
            pltpu.VMEM((N,), jnp.float32),
        ],
        compiler_params=pltpu.CompilerParams(needs_layout_passes=False),
    )


_CH = 80
_NBUF = 4


@functools.lru_cache(maxsize=None)
def _agg_call(E, N, D):
    epw = E // NW
    nch = epw // _CH
    ng = nch // _NBUF
    rem = nch - ng * _NBUF
    assert rem <= 2
    Npad = (N // 128 + 1) * 128
    rpt = Npad // NS

    def body(h_hbm, src_hbm, dst_hbm, zz_hbm, out_hbm,
             s0, s1, s2, s3, d0, d1, d2, d3, r0, r1, r2, r3,
             agg_sh, *sems):
        rows = (r0, r1, r2, r3)
        sidx = (s0, s1, s2, s3)
        didx = (d0, d1, d2, d3)
        gsem, ssem, xsem, ysem = sems[0:4], sems[4:8], sems[8:12], sems[12:16]
        cid = lax.axis_index("c")
        sid = lax.axis_index("s")
        wid = cid * NS + sid

        def sload(i, k):
            return pltpu.make_async_copy(
                src_hbm.at[pl.ds(wid * epw + i * _CH, _CH)], sidx[k], xsem[k])

        def dload(i, k):
            return pltpu.make_async_copy(
                dst_hbm.at[pl.ds(wid * epw + i * _CH, _CH)], didx[k], ysem[k])

        def gather(i, k):
            return pltpu.make_async_copy(h_hbm.at[sidx[k]], rows[k], gsem[k])

        def scatter(k):
            return pltpu.make_async_copy(rows[k], agg_sh.at[didx[k]], ssem[k])

        for k in range(_NBUF):
            sload(k, k).start()
        for k in range(2):
            dload(k, k).start()
        for k in range(2):
            sload(k, k).wait()
            gather(k, k).start()
        pltpu.sync_copy(zz_hbm.at[pl.ds(sid * rpt, rpt)],
                        agg_sh.at[pl.ds(sid * rpt, rpt)])
        plsc.subcore_barrier()

        def slot(i, k):
            gather(i, k).wait()
            dload(i, k).wait()
            scatter(k).start(add=True)
            kk = (k + 2) % _NBUF
            @pl.when(i >= 2)
            def _():
                scatter(kk).wait()
            @pl.when(i + 2 < nch)
            def _():
                dload(i + 2, kk).start()
                sload(i + 2, kk).wait()
                gather(i + 2, kk).start()
            @pl.when(i + 4 < nch)
            def _():
                sload(i + 4, k).start()

        def group(g, c):
            for k in range(_NBUF):
                slot(g * _NBUF + k, k)
            return c
        lax.fori_loop(0, ng, group, 0)
        for t in range(rem):
            slot(ng * _NBUF + t, t)

        for i in range(nch - 2, nch):
            scatter(i % _NBUF).wait()
        plsc.subcore_barrier()
        pltpu.sync_copy(agg_sh.at[pl.ds(sid * rpt, rpt)],
                        out_hbm.at[cid, pl.ds(sid * rpt, rpt)])

    return pl.kernel(
        body,
        out_type=jax.ShapeDtypeStruct((NC, Npad, D), jnp.float32),
        mesh=_MESH,
        scratch_types=[pltpu.VMEM((_CH,), jnp.int32)] * 8
        + [pltpu.VMEM((_CH, D), jnp.float32)] * 4
        + [pltpu.VMEM_SHARED((Npad, D), jnp.float32)]
        + [pltpu.SemaphoreType.DMA] * 16,
        compiler_params=pltpu.CompilerParams(needs_layout_passes=False),
    )


def _prep_body(x_ref, hs_ref, hd_ref, h1_ref, ns_ref, nd_ref):
    ds = jnp.sum(hs_ref[...], axis=1, keepdims=True)
    dd = jnp.sum(hd_ref[...], axis=1, keepdims=True)
    ns = jnp.where(ds > 0, lax.rsqrt(ds), 0.0)
    nd = jnp.where(dd > 0, lax.rsqrt(dd), 0.0)
    ns_ref[...] = ns
    nd_ref[...] = nd
    h1_ref[...] = x_ref[...] * ns


@functools.lru_cache(maxsize=None)
def _prep_call(N, D, R=400):
    grid = N // R
    return pl.pallas_call(
        _prep_body,
        grid=(grid,),
        in_specs=[
            pl.BlockSpec((R, D), lambda i: (i, 0)),
            pl.BlockSpec((R, NW), lambda i: (i, 0)),
            pl.BlockSpec((R, NW), lambda i: (i, 0)),
        ],
        out_specs=[
            pl.BlockSpec((R, D), lambda i: (i, 0)),
            pl.BlockSpec((R, 1), lambda i: (i, 0)),
            pl.BlockSpec((R, 1), lambda i: (i, 0)),
        ],
        out_shape=[
            jax.ShapeDtypeStruct((N, D), jnp.float32),
            jax.ShapeDtypeStruct((N, 1), jnp.float32),
            jax.ShapeDtypeStruct((N, 1), jnp.float32),
        ],
    )


def _dense_body(apply_tanh, agg_ref, nd_ref, ns_ref, w_ref, b_ref, out_ref):
    a = (agg_ref[0] + agg_ref[1]) * nd_ref[...]
    y = jnp.dot(a, w_ref[...], preferred_element_type=jnp.float32) + b_ref[...]
    if apply_tanh:
        y = jnp.tanh(y) * ns_ref[...]
    out_ref[...] = y


@functools.lru_cache(maxsize=None)
def _dense_call(N, D, H, apply_tanh, R=400):
    grid = N // R
    return pl.pallas_call(
        functools.partial(_dense_body, apply_tanh),
        grid=(grid,),
        in_specs=[
            pl.BlockSpec((NC, R, D), lambda i: (0, i, 0)),
            pl.BlockSpec((R, 1), lambda i: (i, 0)),
            pl.BlockSpec((R, 1), lambda i: (i, 0)),
            pl.BlockSpec((D, H), lambda i: (0, 0)),
            pl.BlockSpec((1, H), lambda i: (0, 0)),
        ],
        out_specs=pl.BlockSpec((R, H), lambda i: (i, 0)),
        out_shape=jax.ShapeDtypeStruct((N, H), jnp.float32),
    )


def kernel(x, edge_index, W1, b1, W2, b2):
    N, D = x.shape
    H = W1.shape[1]
    E = edge_index.shape[1]
    src = edge_index[0]
    dst = edge_index[1]

    hs, hd = _deg_call(E, N)(src, dst)
    h1, ns, nd = _prep_call(N, D)(x, hs[:, 0, :].T, hd[:, 0, :].T)

    Npad = (N // 128 + 1) * 128
    epw_pad = -(-(E // NW) // _CH) * _CH
    while (epw_pad // _CH) % _NBUF > 2:
        epw_pad += _CH
    Epad = epw_pad * NW
    if Epad != E:
        ppw = epw_pad - E // NW
        iota = jnp.arange(ppw * NW, dtype=jnp.int32).reshape(NW, ppw)
        spad = iota % N
        dpad = N + iota % (Npad - N)
        src_p = jnp.concatenate([src.reshape(NW, -1), spad], axis=1).reshape(-1)
        dst_p = jnp.concatenate([dst.reshape(NW, -1), dpad], axis=1).reshape(-1)
    else:
        src_p, dst_p = src, dst

    zz = jnp.zeros((Npad, D), jnp.float32)
    agg1 = _agg_call(Epad, N, D)(h1, src_p, dst_p, zz)
    h2 = _dense_call(N, D, H, True)(agg1, nd, ns, W1, b1[None])

    agg2 = _agg_call(Epad, N, H)(h2, src_p, dst_p, zz)
    half = D // 2
    W2e = jnp.concatenate([W2[:, half:], -W2[:, :half]], axis=1)
    b2e = jnp.concatenate([b2[half:], -b2[:half]])
    out = _dense_call(N, H, D, False)(agg2, nd, ns, W2e, b2e[None])
    return out


# speedup vs baseline: 3.6397x; 1.0905x over previous
import functools

import jax
import jax.numpy as jnp
from jax import lax
from jax.experimental import pallas as pl
from jax.experimental.pallas import tpu as pltpu
from jax.experimental.pallas import tpu_sc as plsc

NC = 2
NS = 16
NW = NC * NS
L = 16

_MESH = plsc.VectorSubcoreMesh(core_axis_name="c", subcore_axis_name="s")


@functools.lru_cache(maxsize=None)
def _deg_call(E, N):
    epw = E // NW
    nvec = epw // L
    nhz = N // L

    def body(src_hbm, dst_hbm, hs_out, hd_out, idx_v, hist):
        cid = lax.axis_index("c")
        sid = lax.axis_index("s")
        wid = cid * NS + sid
        zeros16 = jnp.zeros((L,), jnp.float32)
        ones16 = jnp.ones((L,), jnp.float32)

        def run(ind_hbm, out_hbm):
            def zero_it(i, c):
                hist[pl.ds(i * L, L)] = zeros16
                return c
            lax.fori_loop(0, nhz, zero_it, 0)
            pltpu.sync_copy(ind_hbm.at[pl.ds(wid * epw, epw)], idx_v)

            def acc(i, c):
                idx = idx_v[pl.ds(i * L, L)]
                plsc.addupdate_scatter(hist, [idx], ones16)
                return c
            lax.fori_loop(0, nvec, acc, 0)
            pltpu.sync_copy(hist, out_hbm.at[wid, 0])

        run(src_hbm, hs_out)
        run(dst_hbm, hd_out)

    return pl.kernel(
        body,
        out_type=[
            jax.ShapeDtypeStruct((NW, 1, N), jnp.float32),
            jax.ShapeDtypeStruct((NW, 1, N), jnp.float32),
        ],
        mesh=_MESH,
        scratch_types=[
            pltpu.VMEM((epw,), jnp.int32),
            pltpu.VMEM((N,), jnp.float32),
        ],
        compiler_params=pltpu.CompilerParams(needs_layout_passes=False),
    )


_CH = 80
_NBUF = 4


@functools.lru_cache(maxsize=None)
def _agg_call(E, N, D):
    epw = E // NW
    nch = epw // _CH
    ng = nch // _NBUF
    rem = nch - ng * _NBUF
    assert rem <= 2
    Npad = (N // 128 + 1) * 128
    rpt = Npad // NS

    def body(h_hbm, src_hbm, dst_hbm, zz_hbm, out_hbm,
             s0, s1, s2, s3, d0, d1, d2, d3, r0, r1, r2, r3,
             agg_sh, *sems):
        rows = (r0, r1, r2, r3)
        sidx = (s0, s1, s2, s3)
        didx = (d0, d1, d2, d3)
        gsem, ssem, xsem, ysem = sems[0:4], sems[4:8], sems[8:12], sems[12:16]
        cid = lax.axis_index("c")
        sid = lax.axis_index("s")
        wid = cid * NS + sid

        def sload(i, k):
            return pltpu.make_async_copy(
                src_hbm.at[pl.ds(wid * epw + i * _CH, _CH)], sidx[k], xsem[k])

        def dload(i, k):
            return pltpu.make_async_copy(
                dst_hbm.at[pl.ds(wid * epw + i * _CH, _CH)], didx[k], ysem[k])

        def gather(i, k):
            return pltpu.make_async_copy(h_hbm.at[sidx[k]], rows[k], gsem[k])

        def scatter(k):
            return pltpu.make_async_copy(rows[k], agg_sh.at[didx[k]], ssem[k])

        for k in range(_NBUF):
            sload(k, k).start()
        for k in range(2):
            dload(k, k).start()
        for k in range(2):
            sload(k, k).wait()
            gather(k, k).start()
        pltpu.sync_copy(zz_hbm.at[pl.ds(sid * rpt, rpt)],
                        agg_sh.at[pl.ds(sid * rpt, rpt)])
        plsc.subcore_barrier()

        def slot(i, k):
            gather(i, k).wait()
            dload(i, k).wait()
            scatter(k).start(add=True)
            kk = (k + 2) % _NBUF
            @pl.when(i >= 2)
            def _():
                scatter(kk).wait()
            @pl.when(i + 2 < nch)
            def _():
                dload(i + 2, kk).start()
                sload(i + 2, kk).wait()
                gather(i + 2, kk).start()
            @pl.when(i + 4 < nch)
            def _():
                sload(i + 4, k).start()

        def group(g, c):
            for k in range(_NBUF):
                slot(g * _NBUF + k, k)
            return c
        lax.fori_loop(0, ng, group, 0)
        for t in range(rem):
            slot(ng * _NBUF + t, t)

        for i in range(nch - 2, nch):
            scatter(i % _NBUF).wait()
        plsc.subcore_barrier()
        pltpu.sync_copy(agg_sh.at[pl.ds(sid * rpt, rpt)],
                        out_hbm.at[cid, pl.ds(sid * rpt, rpt)])

    return pl.kernel(
        body,
        out_type=jax.ShapeDtypeStruct((NC, Npad, D), jnp.float32),
        mesh=_MESH,
        scratch_types=[pltpu.VMEM((_CH,), jnp.int32)] * 8
        + [pltpu.VMEM((_CH, D), jnp.float32)] * 4
        + [pltpu.VMEM_SHARED((Npad, D), jnp.float32)]
        + [pltpu.SemaphoreType.DMA] * 16,
        compiler_params=pltpu.CompilerParams(needs_layout_passes=False),
    )


def _prep_body(x_ref, hs_ref, hd_ref, h1_ref, ns_ref, nd_ref):
    ds = jnp.sum(hs_ref[...], axis=1, keepdims=True)
    dd = jnp.sum(hd_ref[...], axis=1, keepdims=True)
    ns = jnp.where(ds > 0, lax.rsqrt(ds), 0.0)
    nd = jnp.where(dd > 0, lax.rsqrt(dd), 0.0)
    ns_ref[...] = ns
    nd_ref[...] = nd
    h1_ref[...] = x_ref[...] * ns


@functools.lru_cache(maxsize=None)
def _prep_call(N, D, R=2000):
    grid = N // R
    return pl.pallas_call(
        _prep_body,
        grid=(grid,),
        in_specs=[
            pl.BlockSpec((R, D), lambda i: (i, 0)),
            pl.BlockSpec((R, NW), lambda i: (i, 0)),
            pl.BlockSpec((R, NW), lambda i: (i, 0)),
        ],
        out_specs=[
            pl.BlockSpec((R, D), lambda i: (i, 0)),
            pl.BlockSpec((R, 1), lambda i: (i, 0)),
            pl.BlockSpec((R, 1), lambda i: (i, 0)),
        ],
        out_shape=[
            jax.ShapeDtypeStruct((N, D), jnp.float32),
            jax.ShapeDtypeStruct((N, 1), jnp.float32),
            jax.ShapeDtypeStruct((N, 1), jnp.float32),
        ],
    )


def _dense_body(apply_tanh, agg_ref, nd_ref, ns_ref, w_ref, b_ref, out_ref):
    a = (agg_ref[0] + agg_ref[1]) * nd_ref[...]
    y = jnp.dot(a, w_ref[...], preferred_element_type=jnp.float32) + b_ref[...]
    if apply_tanh:
        y = jnp.tanh(y) * ns_ref[...]
    out_ref[...] = y


@functools.lru_cache(maxsize=None)
def _dense_call(N, D, H, apply_tanh, R=2000):
    grid = N // R
    return pl.pallas_call(
        functools.partial(_dense_body, apply_tanh),
        grid=(grid,),
        in_specs=[
            pl.BlockSpec((NC, R, D), lambda i: (0, i, 0)),
            pl.BlockSpec((R, 1), lambda i: (i, 0)),
            pl.BlockSpec((R, 1), lambda i: (i, 0)),
            pl.BlockSpec((D, H), lambda i: (0, 0)),
            pl.BlockSpec((1, H), lambda i: (0, 0)),
        ],
        out_specs=pl.BlockSpec((R, H), lambda i: (i, 0)),
        out_shape=jax.ShapeDtypeStruct((N, H), jnp.float32),
    )


def kernel(x, edge_index, W1, b1, W2, b2):
    N, D = x.shape
    H = W1.shape[1]
    E = edge_index.shape[1]

    Npad = (N // 128 + 1) * 128
    epw_pad = -(-(E // NW) // _CH) * _CH
    while (epw_pad // _CH) % _NBUF > 2:
        epw_pad += _CH
    Epad = epw_pad * NW
    if Epad != E:
        ppw = epw_pad - E // NW
        iota = jnp.arange(ppw * NW, dtype=jnp.int32).reshape(NW, ppw)
        src_p = jnp.concatenate(
            [edge_index[0].reshape(NW, -1), iota % N], axis=1).reshape(-1)
        dst_p = jnp.concatenate(
            [edge_index[1].reshape(NW, -1), N + iota % (Npad - N)], axis=1
        ).reshape(-1)
    else:
        src_p, dst_p = edge_index[0], edge_index[1]

    hs, hd = _deg_call(E, N)(edge_index[0], edge_index[1])
    h1, ns, nd = _prep_call(N, D)(x, hs[:, 0, :].T, hd[:, 0, :].T)

    zz = jnp.zeros((Npad, D), jnp.float32)
    agg1 = _agg_call(Epad, N, D)(h1, src_p, dst_p, zz)
    h2 = _dense_call(N, D, H, True)(agg1, nd, ns, W1, b1[None])

    agg2 = _agg_call(Epad, N, H)(h2, src_p, dst_p, zz)
    half = D // 2
    W2e = jnp.concatenate([W2[:, half:], -W2[:, :half]], axis=1)
    b2e = jnp.concatenate([b2[half:], -b2[:half]])
    out = _dense_call(N, H, D, False)(agg2, nd, ns, W2e, b2e[None])
    return out


# speedup vs baseline: 3.6874x; 1.0131x over previous
import functools

import jax
import jax.numpy as jnp
from jax import lax
from jax.experimental import pallas as pl
from jax.experimental.pallas import tpu as pltpu
from jax.experimental.pallas import tpu_sc as plsc

NC = 2
NS = 16
NW = NC * NS
L = 16

_MESH = plsc.VectorSubcoreMesh(core_axis_name="c", subcore_axis_name="s")


@functools.lru_cache(maxsize=None)
def _deg_call(E, N):
    epw = E // NW
    nvec = epw // L
    nhz = N // L

    def body(src_hbm, dst_hbm, hs_out, hd_out, idx_v, hist):
        cid = lax.axis_index("c")
        sid = lax.axis_index("s")
        wid = cid * NS + sid
        zeros16 = jnp.zeros((L,), jnp.float32)
        ones16 = jnp.ones((L,), jnp.float32)

        def run(ind_hbm, out_hbm):
            def zero_it(i, c):
                hist[pl.ds(i * L, L)] = zeros16
                return c
            lax.fori_loop(0, nhz, zero_it, 0)
            pltpu.sync_copy(ind_hbm.at[pl.ds(wid * epw, epw)], idx_v)

            def acc(i, c):
                idx = idx_v[pl.ds(i * L, L)]
                plsc.addupdate_scatter(hist, [idx], ones16)
                return c
            lax.fori_loop(0, nvec, acc, 0)
            pltpu.sync_copy(hist, out_hbm.at[wid, 0])

        run(src_hbm, hs_out)
        run(dst_hbm, hd_out)

    return pl.kernel(
        body,
        out_type=[
            jax.ShapeDtypeStruct((NW, 1, N), jnp.float32),
            jax.ShapeDtypeStruct((NW, 1, N), jnp.float32),
        ],
        mesh=_MESH,
        scratch_types=[
            pltpu.VMEM((epw,), jnp.int32),
            pltpu.VMEM((N,), jnp.float32),
        ],
        compiler_params=pltpu.CompilerParams(needs_layout_passes=False),
    )


_CH = 96
_NBUF = 4


@functools.lru_cache(maxsize=None)
def _agg_call(E, N, D):
    epw = E // NW
    nch = epw // _CH
    ng = nch // _NBUF
    rem = nch - ng * _NBUF
    assert rem <= 2
    Npad = (N // 128 + 1) * 128
    rpt = Npad // NS

    def body(h_hbm, src_hbm, dst_hbm, zz_hbm, out_hbm,
             s0, s1, s2, s3, d0, d1, d2, d3, r0, r1, r2, r3,
             agg_sh, *sems):
        rows = (r0, r1, r2, r3)
        sidx = (s0, s1, s2, s3)
        didx = (d0, d1, d2, d3)
        gsem, ssem, xsem, ysem = sems[0:4], sems[4:8], sems[8:12], sems[12:16]
        cid = lax.axis_index("c")
        sid = lax.axis_index("s")
        wid = cid * NS + sid

        def sload(i, k):
            return pltpu.make_async_copy(
                src_hbm.at[pl.ds(wid * epw + i * _CH, _CH)], sidx[k], xsem[k])

        def dload(i, k):
            return pltpu.make_async_copy(
                dst_hbm.at[pl.ds(wid * epw + i * _CH, _CH)], didx[k], ysem[k])

        def gather(i, k):
            return pltpu.make_async_copy(h_hbm.at[sidx[k]], rows[k], gsem[k])

        def scatter(k):
            return pltpu.make_async_copy(rows[k], agg_sh.at[didx[k]], ssem[k])

        for k in range(_NBUF):
            sload(k, k).start()
        for k in range(2):
            dload(k, k).start()
        for k in range(2):
            sload(k, k).wait()
            gather(k, k).start()
        pltpu.sync_copy(zz_hbm.at[pl.ds(sid * rpt, rpt)],
                        agg_sh.at[pl.ds(sid * rpt, rpt)])
        plsc.subcore_barrier()

        def slot(i, k):
            gather(i, k).wait()
            dload(i, k).wait()
            scatter(k).start(add=True)
            kk = (k + 2) % _NBUF
            @pl.when(i >= 2)
            def _():
                scatter(kk).wait()
            @pl.when(i + 2 < nch)
            def _():
                dload(i + 2, kk).start()
                sload(i + 2, kk).wait()
                gather(i + 2, kk).start()
            @pl.when(i + 4 < nch)
            def _():
                sload(i + 4, k).start()

        def group(g, c):
            for k in range(_NBUF):
                slot(g * _NBUF + k, k)
            return c
        lax.fori_loop(0, ng, group, 0)
        for t in range(rem):
            slot(ng * _NBUF + t, t)

        for i in range(nch - 2, nch):
            scatter(i % _NBUF).wait()
        plsc.subcore_barrier()
        pltpu.sync_copy(agg_sh.at[pl.ds(sid * rpt, rpt)],
                        out_hbm.at[cid, pl.ds(sid * rpt, rpt)])

    return pl.kernel(
        body,
        out_type=jax.ShapeDtypeStruct((NC, Npad, D), jnp.float32),
        mesh=_MESH,
        scratch_types=[pltpu.VMEM((_CH,), jnp.int32)] * 8
        + [pltpu.VMEM((_CH, D), jnp.float32)] * 4
        + [pltpu.VMEM_SHARED((Npad, D), jnp.float32)]
        + [pltpu.SemaphoreType.DMA] * 16,
        compiler_params=pltpu.CompilerParams(needs_layout_passes=False),
    )


def _prep_body(x_ref, hs_ref, hd_ref, h1_ref, ns_ref, nd_ref):
    ds = jnp.sum(hs_ref[...], axis=1, keepdims=True)
    dd = jnp.sum(hd_ref[...], axis=1, keepdims=True)
    ns = jnp.where(ds > 0, lax.rsqrt(ds), 0.0)
    nd = jnp.where(dd > 0, lax.rsqrt(dd), 0.0)
    ns_ref[...] = ns
    nd_ref[...] = nd
    h1_ref[...] = x_ref[...] * ns


@functools.lru_cache(maxsize=None)
def _prep_call(N, D, R=2000):
    grid = N // R
    return pl.pallas_call(
        _prep_body,
        grid=(grid,),
        in_specs=[
            pl.BlockSpec((R, D), lambda i: (i, 0)),
            pl.BlockSpec((R, NW), lambda i: (i, 0)),
            pl.BlockSpec((R, NW), lambda i: (i, 0)),
        ],
        out_specs=[
            pl.BlockSpec((R, D), lambda i: (i, 0)),
            pl.BlockSpec((R, 1), lambda i: (i, 0)),
            pl.BlockSpec((R, 1), lambda i: (i, 0)),
        ],
        out_shape=[
            jax.ShapeDtypeStruct((N, D), jnp.float32),
            jax.ShapeDtypeStruct((N, 1), jnp.float32),
            jax.ShapeDtypeStruct((N, 1), jnp.float32),
        ],
    )


def _dense_body(apply_tanh, agg_ref, nd_ref, ns_ref, w_ref, b_ref, out_ref):
    a = (agg_ref[0] + agg_ref[1]) * nd_ref[...]
    y = jnp.dot(a, w_ref[...], preferred_element_type=jnp.float32) + b_ref[...]
    if apply_tanh:
        y = jnp.tanh(y) * ns_ref[...]
    out_ref[...] = y


@functools.lru_cache(maxsize=None)
def _dense_call(N, D, H, apply_tanh, R=2000):
    grid = N // R
    return pl.pallas_call(
        functools.partial(_dense_body, apply_tanh),
        grid=(grid,),
        in_specs=[
            pl.BlockSpec((NC, R, D), lambda i: (0, i, 0)),
            pl.BlockSpec((R, 1), lambda i: (i, 0)),
            pl.BlockSpec((R, 1), lambda i: (i, 0)),
            pl.BlockSpec((D, H), lambda i: (0, 0)),
            pl.BlockSpec((1, H), lambda i: (0, 0)),
        ],
        out_specs=pl.BlockSpec((R, H), lambda i: (i, 0)),
        out_shape=jax.ShapeDtypeStruct((N, H), jnp.float32),
    )


def kernel(x, edge_index, W1, b1, W2, b2):
    N, D = x.shape
    H = W1.shape[1]
    E = edge_index.shape[1]

    Npad = (N // 128 + 1) * 128
    epw_pad = -(-(E // NW) // _CH) * _CH
    while (epw_pad // _CH) % _NBUF > 2:
        epw_pad += _CH
    Epad = epw_pad * NW
    if Epad != E:
        ppw = epw_pad - E // NW
        iota = jnp.arange(ppw * NW, dtype=jnp.int32).reshape(NW, ppw)
        src_p = jnp.concatenate(
            [edge_index[0].reshape(NW, -1), iota % N], axis=1).reshape(-1)
        dst_p = jnp.concatenate(
            [edge_index[1].reshape(NW, -1), N + iota % (Npad - N)], axis=1
        ).reshape(-1)
    else:
        src_p, dst_p = edge_index[0], edge_index[1]

    hs, hd = _deg_call(E, N)(edge_index[0], edge_index[1])
    h1, ns, nd = _prep_call(N, D)(x, hs[:, 0, :].T, hd[:, 0, :].T)

    zz = jnp.zeros((Npad, D), jnp.float32)
    agg1 = _agg_call(Epad, N, D)(h1, src_p, dst_p, zz)
    h2 = _dense_call(N, D, H, True)(agg1, nd, ns, W1, b1[None])

    agg2 = _agg_call(Epad, N, H)(h2, src_p, dst_p, zz)
    half = D // 2
    W2e = jnp.concatenate([W2[:, half:], -W2[:, :half]], axis=1)
    b2e = jnp.concatenate([b2[half:], -b2[:half]])
    out = _dense_call(N, H, D, False)(agg2, nd, ns, W2e, b2e[None])
    return out
